# Initial kernel scaffold; baseline (speedup 1.0000x reference)
import functools

import jax
import jax.numpy as jnp
from jax import lax
from jax.experimental import pallas as pl
from jax.experimental.pallas import tpu as pltpu
from jax.experimental.pallas import tpu_sc as plsc

B = 16384
L = 50
D = 64
NG = 2 * B
NW = 32
GPW = NG // NW
IB = 64
NB = GPW // IB
LANES = 16
NV = D // LANES
INV_L = 1.0 / L


def _pool_body(idx_hbm, table_hbm, out_hbm, idx_v, rows0, rows1, outblk, sem0, sem1):
    wid = lax.axis_index("s") * 2 + lax.axis_index("c")
    base = wid * GPW

    def accumulate(rows, g):
        for j in range(NV):
            a = rows[0, pl.ds(j * LANES, LANES)]
            for r in range(1, L):
                a = a + rows[r, pl.ds(j * LANES, LANES)]
            outblk[g, pl.ds(j * LANES, LANES)] = a * INV_L

    def block_body(blk):
        row0 = base + blk * IB
        pltpu.sync_copy(idx_hbm.at[pl.ds(row0, IB)], idx_v)
        pltpu.async_copy(table_hbm.at[idx_v.at[0]], rows0, sem0)

        def pair_body(p):
            g0 = 2 * p
            g1 = g0 + 1
            cp1 = pltpu.async_copy(table_hbm.at[idx_v.at[g1]], rows1, sem1)
            pltpu.make_async_copy(table_hbm.at[idx_v.at[g0]], rows0, sem0).wait()
            accumulate(rows0, g0)

            @pl.when(g0 + 2 < IB)
            def _():
                pltpu.async_copy(table_hbm.at[idx_v.at[g0 + 2]], rows0, sem0)

            cp1.wait()
            accumulate(rows1, g1)

        pl.loop(0, IB // 2)(pair_body)
        pltpu.sync_copy(outblk, out_hbm.at[pl.ds(row0, IB)])

    pl.loop(0, NB)(block_body)


@jax.jit
def _pooled_lookup(idx, table):
    mesh = plsc.VectorSubcoreMesh(core_axis_name="c", subcore_axis_name="s")
    return pl.kernel(
        _pool_body,
        out_type=jax.ShapeDtypeStruct((NG, D), jnp.float32),
        mesh=mesh,
        scratch_types=[
            pltpu.VMEM((IB, L), jnp.int32),
            pltpu.VMEM((L, D), jnp.float32),
            pltpu.VMEM((L, D), jnp.float32),
            pltpu.VMEM((IB, D), jnp.float32),
            pltpu.SemaphoreType.DMA,
            pltpu.SemaphoreType.DMA,
        ],
    )(idx, table)


def _mm_body(x_ref, w_ref, b_ref, o_ref):
    o_ref[...] = (
        jnp.dot(x_ref[...], w_ref[...], preferred_element_type=jnp.float32)
        + b_ref[...]
    )


@jax.jit
def _project(pooled, W, b):
    blk = 4096
    return pl.pallas_call(
        _mm_body,
        grid=(NG // blk,),
        in_specs=[
            pl.BlockSpec((blk, D), lambda i: (i, 0)),
            pl.BlockSpec((D, D), lambda i: (0, 0)),
            pl.BlockSpec((1, D), lambda i: (0, 0)),
        ],
        out_specs=pl.BlockSpec((blk, D), lambda i: (i, 0)),
        out_shape=jax.ShapeDtypeStruct((NG, D), jnp.float32),
    )(pooled, W, b.reshape(1, D))


def kernel(query, candidate, table, W, b):
    idx = jnp.concatenate([query, candidate], axis=0).astype(jnp.int32)
    pooled = _pooled_lookup(idx, table)
    out = _project(pooled, W, b)
    return (out[:B], out[B:])


# speedup vs baseline: 3.0479x; 3.0479x over previous
import functools

import jax
import jax.numpy as jnp
from jax import lax
from jax.experimental import pallas as pl
from jax.experimental.pallas import tpu as pltpu
from jax.experimental.pallas import tpu_sc as plsc

B = 16384
L = 50
D = 64
NG = 2 * B
NW = 32
GPW = NG // NW
IB = 64
NB = GPW // IB
LANES = 16
NV = D // LANES
INV_L = 1.0 / L


def _pool_body(idx_hbm, table_hbm, out_hbm, idx_v, rows0, rows1, outblk, sem0, sem1):
    wid = lax.axis_index("s") * 2 + lax.axis_index("c")
    base = wid * GPW

    def accumulate(rows, g):
        for j in range(NV):
            a = rows[0, pl.ds(j * LANES, LANES)]
            for r in range(1, L):
                a = a + rows[r, pl.ds(j * LANES, LANES)]
            outblk[g, pl.ds(j * LANES, LANES)] = a * INV_L

    def block_body(blk):
        row0 = base + blk * IB
        pltpu.sync_copy(idx_hbm.at[pl.ds(row0, IB)], idx_v)
        pltpu.async_copy(table_hbm.at[idx_v.at[0]], rows0, sem0)

        def pair_body(p):
            g0 = 2 * p
            g1 = g0 + 1
            cp1 = pltpu.async_copy(table_hbm.at[idx_v.at[g1]], rows1, sem1)
            pltpu.make_async_copy(table_hbm.at[idx_v.at[g0]], rows0, sem0).wait()
            accumulate(rows0, g0)

            @pl.when(g0 + 2 < IB)
            def _():
                pltpu.async_copy(table_hbm.at[idx_v.at[g0 + 2]], rows0, sem0)

            cp1.wait()
            accumulate(rows1, g1)

        pl.loop(0, IB // 2)(pair_body)
        pltpu.sync_copy(outblk, out_hbm.at[pl.ds(row0, IB)])

    pl.loop(0, NB)(block_body)


@jax.jit
def _pooled_lookup(idx, table):
    mesh = plsc.VectorSubcoreMesh(core_axis_name="c", subcore_axis_name="s")
    return pl.kernel(
        _pool_body,
        out_type=jax.ShapeDtypeStruct((NG, D), jnp.float32),
        mesh=mesh,
        scratch_types=[
            pltpu.VMEM((IB, L), jnp.int32),
            pltpu.VMEM((L, D), jnp.float32),
            pltpu.VMEM((L, D), jnp.float32),
            pltpu.VMEM((IB, D), jnp.float32),
            pltpu.SemaphoreType.DMA,
            pltpu.SemaphoreType.DMA,
        ],
        compiler_params=pltpu.CompilerParams(use_tc_tiling_on_sc=False),
    )(idx, table)


def _mm_body(x_ref, w_ref, b_ref, o_ref):
    o_ref[...] = (
        jnp.dot(x_ref[...], w_ref[...], preferred_element_type=jnp.float32)
        + b_ref[...]
    )


@jax.jit
def _project(pooled, W, b):
    blk = 4096
    return pl.pallas_call(
        _mm_body,
        grid=(NG // blk,),
        in_specs=[
            pl.BlockSpec((blk, D), lambda i: (i, 0)),
            pl.BlockSpec((D, D), lambda i: (0, 0)),
            pl.BlockSpec((1, D), lambda i: (0, 0)),
        ],
        out_specs=pl.BlockSpec((blk, D), lambda i: (i, 0)),
        out_shape=jax.ShapeDtypeStruct((NG, D), jnp.float32),
    )(pooled, W, b.reshape(1, D))


def kernel(query, candidate, table, W, b):
    idx = jnp.concatenate([query, candidate], axis=0).astype(jnp.int32)
    pooled = _pooled_lookup(idx, table)
    out = _project(pooled, W, b)
    return (out[:B], out[B:])
